# baseline (speedup 1.0000x reference)
import functools

import jax
import jax.numpy as jnp
from jax import lax
from jax.experimental import pallas as pl
from jax.experimental.pallas import tpu as pltpu
from jax.experimental.pallas import tpu_sc as plsc

B = 16384
V = 1000000
D = 16
NC = 2
NS = 16
NW = NC * NS
NPW = B // NW
L = 16
CH = 128
NCH = NPW // CH


def _sc_body(uidx_hbm, iidx_hbm, eu_hbm, ei_hbm, bu_hbm, bi_hbm, gb_hbm,
             out_hbm,
             uidx_v, iidx_v, eu_v, ei_v, bu_v, bi_v, gb_v, out_v, sem):
    wid = lax.axis_index("s") * NC + lax.axis_index("c")
    base = wid * NPW

    pltpu.sync_copy(uidx_hbm.at[pl.ds(base, NPW)], uidx_v)
    pltpu.sync_copy(iidx_hbm.at[pl.ds(base, NPW)], iidx_v)
    pltpu.sync_copy(gb_hbm, gb_v)

    for c in range(NCH):
        s = pl.ds(c * CH, CH)
        for d in range(D):
            dst = pl.ds(d * NPW + c * CH, CH)
            pltpu.async_copy(
                eu_hbm.at[d].at[uidx_v.at[s]], eu_v.at[dst], sem)
            pltpu.async_copy(
                ei_hbm.at[d].at[iidx_v.at[s]], ei_v.at[dst], sem)
        pltpu.async_copy(bu_hbm.at[uidx_v.at[s]], bu_v.at[s], sem)
        pltpu.async_copy(bi_hbm.at[iidx_v.at[s]], bi_v.at[s], sem)

    pltpu.make_async_copy(uidx_hbm.at[pl.ds(0, D * NPW)], eu_v, sem).wait()
    pltpu.make_async_copy(uidx_hbm.at[pl.ds(0, D * NPW)], ei_v, sem).wait()
    pltpu.make_async_copy(uidx_hbm.at[pl.ds(0, NPW)], bu_v, sem).wait()
    pltpu.make_async_copy(uidx_hbm.at[pl.ds(0, NPW)], bi_v, sem).wait()

    gb = gb_v[...]

    def block(t, _):
        p = t * L
        acc = bu_v[pl.ds(p, L)] + bi_v[pl.ds(p, L)] + gb
        for d in range(D):
            acc = acc + eu_v[pl.ds(d * NPW + p, L)] * ei_v[pl.ds(d * NPW + p, L)]
        out_v[pl.ds(p, L)] = jnp.minimum(jnp.maximum(acc, 1.0), 5.0)
        return 0

    lax.fori_loop(0, NPW // L, block, 0)

    pltpu.sync_copy(out_v, out_hbm.at[pl.ds(base, NPW)])


@jax.jit
def _funk_svd_sc(uidx, iidx, emb_ut, emb_it, bias_u, bias_i, gb16):
    mesh = plsc.VectorSubcoreMesh(
        core_axis_name="c", subcore_axis_name="s",
        num_cores=NC, num_subcores=NS)
    run = pl.kernel(
        _sc_body,
        out_type=jax.ShapeDtypeStruct((B,), jnp.float32),
        mesh=mesh,
        scratch_types=[
            pltpu.VMEM((NPW,), jnp.int32),
            pltpu.VMEM((NPW,), jnp.int32),
            pltpu.VMEM((D * NPW,), jnp.float32),
            pltpu.VMEM((D * NPW,), jnp.float32),
            pltpu.VMEM((NPW,), jnp.float32),
            pltpu.VMEM((NPW,), jnp.float32),
            pltpu.VMEM((L,), jnp.float32),
            pltpu.VMEM((NPW,), jnp.float32),
            pltpu.SemaphoreType.DMA,
        ],
        compiler_params=pltpu.CompilerParams(
            needs_layout_passes=False,
            use_tc_tiling_on_sc=False,
        ),
    )
    return run(uidx, iidx, emb_ut, emb_it, bias_u, bias_i, gb16)


def kernel(user_idx, item_idx, emb_u, emb_i, bias_u, bias_i, global_bias):
    uidx = user_idx.astype(jnp.int32)
    iidx = item_idx.astype(jnp.int32)
    gb16 = jnp.broadcast_to(global_bias.astype(jnp.float32), (L,))
    return _funk_svd_sc(uidx, iidx, emb_u.T, emb_i.T, bias_u, bias_i, gb16)


# speedup vs baseline: 3.7894x; 3.7894x over previous
import functools

import jax
import jax.numpy as jnp
from jax import lax
from jax.experimental import pallas as pl
from jax.experimental.pallas import tpu as pltpu
from jax.experimental.pallas import tpu_sc as plsc

B = 16384
V = 1000000
D = 16
NC = 2
NS = 16
NW = NC * NS
NPW = B // NW
L = 16
CH = 128
NCH = NPW // CH


def _sc_body(uidx_hbm, iidx_hbm, *rest):
    cu = rest[:D]
    ci = rest[D:2 * D]
    (bu_hbm, bi_hbm, gb_hbm, out_hbm,
     uidx_v, iidx_v, eu_v, ei_v, bu_v, bi_v, gb_v, out_v, sem) = rest[2 * D:]

    wid = lax.axis_index("s") * NC + lax.axis_index("c")
    base = wid * NPW

    pltpu.sync_copy(uidx_hbm.at[pl.ds(base, NPW)], uidx_v)
    pltpu.sync_copy(iidx_hbm.at[pl.ds(base, NPW)], iidx_v)
    pltpu.sync_copy(gb_hbm, gb_v)

    for c in range(NCH):
        s = pl.ds(c * CH, CH)
        for d in range(D):
            dst = pl.ds(d * NPW + c * CH, CH)
            pltpu.async_copy(cu[d].at[uidx_v.at[s]], eu_v.at[dst], sem)
            pltpu.async_copy(ci[d].at[iidx_v.at[s]], ei_v.at[dst], sem)
        pltpu.async_copy(bu_hbm.at[uidx_v.at[s]], bu_v.at[s], sem)
        pltpu.async_copy(bi_hbm.at[iidx_v.at[s]], bi_v.at[s], sem)

    pltpu.make_async_copy(uidx_hbm.at[pl.ds(0, D * NPW)], eu_v, sem).wait()
    pltpu.make_async_copy(uidx_hbm.at[pl.ds(0, D * NPW)], ei_v, sem).wait()
    pltpu.make_async_copy(uidx_hbm.at[pl.ds(0, NPW)], bu_v, sem).wait()
    pltpu.make_async_copy(uidx_hbm.at[pl.ds(0, NPW)], bi_v, sem).wait()

    gb = gb_v[...]

    def block(t, _):
        p = t * L
        acc = bu_v[pl.ds(p, L)] + bi_v[pl.ds(p, L)] + gb
        for d in range(D):
            acc = acc + eu_v[pl.ds(d * NPW + p, L)] * ei_v[pl.ds(d * NPW + p, L)]
        out_v[pl.ds(p, L)] = jnp.minimum(jnp.maximum(acc, 1.0), 5.0)
        return 0

    lax.fori_loop(0, NPW // L, block, 0)

    pltpu.sync_copy(out_v, out_hbm.at[pl.ds(base, NPW)])


@jax.jit
def _funk_svd_sc(uidx, iidx, cols_u, cols_i, bias_u, bias_i, gb16):
    mesh = plsc.VectorSubcoreMesh(
        core_axis_name="c", subcore_axis_name="s",
        num_cores=NC, num_subcores=NS)
    run = pl.kernel(
        _sc_body,
        out_type=jax.ShapeDtypeStruct((B,), jnp.float32),
        mesh=mesh,
        scratch_types=[
            pltpu.VMEM((NPW,), jnp.int32),
            pltpu.VMEM((NPW,), jnp.int32),
            pltpu.VMEM((D * NPW,), jnp.float32),
            pltpu.VMEM((D * NPW,), jnp.float32),
            pltpu.VMEM((NPW,), jnp.float32),
            pltpu.VMEM((NPW,), jnp.float32),
            pltpu.VMEM((L,), jnp.float32),
            pltpu.VMEM((NPW,), jnp.float32),
            pltpu.SemaphoreType.DMA,
        ],
        compiler_params=pltpu.CompilerParams(
            needs_layout_passes=False,
            use_tc_tiling_on_sc=False,
        ),
    )
    return run(uidx, iidx, *cols_u, *cols_i, bias_u, bias_i, gb16)


def kernel(user_idx, item_idx, emb_u, emb_i, bias_u, bias_i, global_bias):
    uidx = user_idx.astype(jnp.int32)
    iidx = item_idx.astype(jnp.int32)
    gb16 = jnp.broadcast_to(global_bias.astype(jnp.float32), (L,))
    cols_u = tuple(jax.lax.squeeze(
        jax.lax.slice_in_dim(emb_u, d, d + 1, axis=1), (1,)) for d in range(D))
    cols_i = tuple(jax.lax.squeeze(
        jax.lax.slice_in_dim(emb_i, d, d + 1, axis=1), (1,)) for d in range(D))
    return _funk_svd_sc(uidx, iidx, cols_u, cols_i, bias_u, bias_i, gb16)


# speedup vs baseline: 17.8469x; 4.7097x over previous
import functools

import jax
import jax.numpy as jnp
from jax import lax
from jax.experimental import pallas as pl
from jax.experimental.pallas import tpu as pltpu
from jax.experimental.pallas import tpu_sc as plsc

B = 16384
V = 1000000
D = 16
NC = 2
NS = 16
NW = NC * NS
NPW = B // NW
L = 16
CH = 128
NCH = NPW // CH

CHW = 65536
NFULL = 15
TAIL_A = NFULL * CHW
TAIL_N = 16896
PAD_A = TAIL_A + TAIL_N
PAD_N = V - PAD_A


def _detile_body(tu_hbm, ti_hbm, flat_hbm, buf, sem):
    wid = lax.axis_index("s") * NC + lax.axis_index("c")
    t = wid // D
    d = wid % D
    g = d // 8
    s = d % 8
    obase = wid * V

    def column(src3):
        view = src3.at[g, s]

        def chunk(k, _):
            a = k * CHW
            pltpu.async_copy(view.at[pl.ds(a, CHW)], buf.at[pl.ds(0, CHW)],
                             sem).wait()
            pltpu.async_copy(buf.at[pl.ds(0, CHW)],
                             flat_hbm.at[pl.ds(obase + a, CHW)], sem).wait()
            return 0

        lax.fori_loop(0, NFULL, chunk, 0)
        pltpu.async_copy(view.at[pl.ds(TAIL_A + wid * 0, TAIL_N)],
                         buf.at[pl.ds(0, TAIL_N)], sem).wait()
        pltpu.async_copy(view.at[pl.ds(PAD_A + wid * 0, 128)],
                         buf.at[pl.ds(TAIL_N, 128)], sem).wait()
        pltpu.async_copy(buf.at[pl.ds(0, TAIL_N + PAD_N)],
                         flat_hbm.at[pl.ds(obase + TAIL_A, TAIL_N + PAD_N)],
                         sem).wait()

    @pl.when(t == 0)
    def _():
        column(tu_hbm.reshape(2, 8, V))

    @pl.when(t == 1)
    def _():
        column(ti_hbm.reshape(2, 8, V))


def _gather_body(uidx_hbm, iidx_hbm, flat_hbm, bu_hbm, bi_hbm, gb_hbm,
                 out_hbm,
                 uidx_v, iidx_v, offu_v, offi_v, eu_v, ei_v, bu_v, bi_v,
                 gb_v, out_v, sem):
    wid = lax.axis_index("s") * NC + lax.axis_index("c")
    base = wid * NPW

    pltpu.sync_copy(uidx_hbm.at[pl.ds(base, NPW)], uidx_v)
    pltpu.sync_copy(iidx_hbm.at[pl.ds(base, NPW)], iidx_v)
    pltpu.sync_copy(gb_hbm, gb_v)

    def gen_offsets(tt, _):
        p = tt * L
        vu = uidx_v[pl.ds(p, L)]
        vi = iidx_v[pl.ds(p, L)]
        for d in range(D):
            offu_v[pl.ds(d * NPW + p, L)] = vu + (d * V)
            offi_v[pl.ds(d * NPW + p, L)] = vi + (D * V + d * V)
        return 0

    lax.fori_loop(0, NPW // L, gen_offsets, 0)

    for c in range((D * NPW) // CH):
        s = pl.ds(c * CH, CH)
        pltpu.async_copy(flat_hbm.at[offu_v.at[s]], eu_v.at[s], sem)
        pltpu.async_copy(flat_hbm.at[offi_v.at[s]], ei_v.at[s], sem)
    for c in range(NCH):
        s = pl.ds(c * CH, CH)
        pltpu.async_copy(bu_hbm.at[uidx_v.at[s]], bu_v.at[s], sem)
        pltpu.async_copy(bi_hbm.at[iidx_v.at[s]], bi_v.at[s], sem)

    pltpu.make_async_copy(uidx_hbm.at[pl.ds(0, D * NPW)], eu_v, sem).wait()
    pltpu.make_async_copy(uidx_hbm.at[pl.ds(0, D * NPW)], ei_v, sem).wait()
    pltpu.make_async_copy(uidx_hbm.at[pl.ds(0, NPW)], bu_v, sem).wait()
    pltpu.make_async_copy(uidx_hbm.at[pl.ds(0, NPW)], bi_v, sem).wait()

    gb = gb_v[...]

    def block(tt, _):
        p = tt * L
        acc = bu_v[pl.ds(p, L)] + bi_v[pl.ds(p, L)] + gb
        for d in range(D):
            acc = acc + eu_v[pl.ds(d * NPW + p, L)] * ei_v[pl.ds(d * NPW + p, L)]
        out_v[pl.ds(p, L)] = jnp.minimum(jnp.maximum(acc, 1.0), 5.0)
        return 0

    lax.fori_loop(0, NPW // L, block, 0)

    pltpu.sync_copy(out_v, out_hbm.at[pl.ds(base, NPW)])


@jax.jit
def _funk_svd_sc(uidx, iidx, emb_ut, emb_it, bias_u, bias_i, gb16):
    mesh = plsc.VectorSubcoreMesh(
        core_axis_name="c", subcore_axis_name="s",
        num_cores=NC, num_subcores=NS)

    detile = pl.kernel(
        _detile_body,
        out_type=jax.ShapeDtypeStruct((2 * D * V,), jnp.float32),
        mesh=mesh,
        scratch_types=[
            pltpu.VMEM((CHW,), jnp.float32),
            pltpu.SemaphoreType.DMA,
        ],
        compiler_params=pltpu.CompilerParams(
            needs_layout_passes=False,
            disable_bounds_checks=True,
        ),
    )
    flat = detile(emb_ut, emb_it)

    gather = pl.kernel(
        _gather_body,
        out_type=jax.ShapeDtypeStruct((B,), jnp.float32),
        mesh=mesh,
        scratch_types=[
            pltpu.VMEM((NPW,), jnp.int32),
            pltpu.VMEM((NPW,), jnp.int32),
            pltpu.VMEM((D * NPW,), jnp.int32),
            pltpu.VMEM((D * NPW,), jnp.int32),
            pltpu.VMEM((D * NPW,), jnp.float32),
            pltpu.VMEM((D * NPW,), jnp.float32),
            pltpu.VMEM((NPW,), jnp.float32),
            pltpu.VMEM((NPW,), jnp.float32),
            pltpu.VMEM((L,), jnp.float32),
            pltpu.VMEM((NPW,), jnp.float32),
            pltpu.SemaphoreType.DMA,
        ],
        compiler_params=pltpu.CompilerParams(
            needs_layout_passes=False,
            use_tc_tiling_on_sc=False,
        ),
    )
    return gather(uidx, iidx, flat, bias_u, bias_i, gb16)


def kernel(user_idx, item_idx, emb_u, emb_i, bias_u, bias_i, global_bias):
    uidx = user_idx.astype(jnp.int32)
    iidx = item_idx.astype(jnp.int32)
    gb16 = jnp.broadcast_to(global_bias.astype(jnp.float32), (L,))
    return _funk_svd_sc(uidx, iidx, emb_u.T, emb_i.T, bias_u, bias_i, gb16)


# speedup vs baseline: 18.9439x; 1.0615x over previous
import functools

import jax
import jax.numpy as jnp
from jax import lax
from jax.experimental import pallas as pl
from jax.experimental.pallas import tpu as pltpu
from jax.experimental.pallas import tpu_sc as plsc

B = 16384
V = 1000000
D = 16
NC = 2
NS = 16
NW = NC * NS
NPW = B // NW
L = 16
CH = 128
NCH = NPW // CH

CHW = 61440
NFULL = 16
TAIL_A = NFULL * CHW
TAIL_N = 16896
PAD_A = TAIL_A + TAIL_N
PAD_N = V - PAD_A


def _detile_body(tu_hbm, ti_hbm, flat_hbm, buf, sem_l, sem_r):
    wid = lax.axis_index("s") * NC + lax.axis_index("c")
    t = wid // D
    d = wid % D
    g = d // 8
    s = d % 8
    obase = wid * V

    def column(src3):
        view = src3.at[g, s]
        half = [pl.ds(0, CHW), pl.ds(CHW, CHW)]
        hl = [None] * (NFULL + 1)
        hs = [None] * NFULL
        hl[0] = pltpu.async_copy(view.at[pl.ds(0, CHW)], buf.at[half[0]],
                                 sem_l)
        tb = (NFULL % 2) * CHW
        tl1 = tl2 = None
        for k in range(NFULL):
            cur = half[k % 2]
            hl[k].wait()
            if k >= 1:
                hs[k - 1].wait()
            if k + 1 < NFULL:
                hl[k + 1] = pltpu.async_copy(
                    view.at[pl.ds((k + 1) * CHW, CHW)], buf.at[half[(k + 1) % 2]],
                    sem_l)
            else:
                tl1 = pltpu.async_copy(
                    view.at[pl.ds(TAIL_A + wid * 0, TAIL_N)],
                    buf.at[pl.ds(tb, TAIL_N)], sem_l)
                tl2 = pltpu.async_copy(
                    view.at[pl.ds(PAD_A + wid * 0, 128)],
                    buf.at[pl.ds(tb + TAIL_N, 128)], sem_l)
            hs[k] = pltpu.async_copy(
                buf.at[cur], flat_hbm.at[pl.ds(obase + k * CHW, CHW)], sem_r)
        tl1.wait()
        tl2.wait()
        hs[NFULL - 1].wait()
        pltpu.async_copy(
            buf.at[pl.ds(tb, TAIL_N + PAD_N)],
            flat_hbm.at[pl.ds(obase + TAIL_A, TAIL_N + PAD_N)], sem_r).wait()

    @pl.when(t == 0)
    def _():
        column(tu_hbm.reshape(2, 8, V))

    @pl.when(t == 1)
    def _():
        column(ti_hbm.reshape(2, 8, V))


def _gather_body(uidx_hbm, iidx_hbm, flat_hbm, bu_hbm, bi_hbm, gb_hbm,
                 out_hbm,
                 uidx_v, iidx_v, offu_v, offi_v, eu_v, ei_v, bu_v, bi_v,
                 gb_v, out_v, sem):
    wid = lax.axis_index("s") * NC + lax.axis_index("c")
    base = wid * NPW

    pltpu.sync_copy(uidx_hbm.at[pl.ds(base, NPW)], uidx_v)
    pltpu.sync_copy(iidx_hbm.at[pl.ds(base, NPW)], iidx_v)
    pltpu.sync_copy(gb_hbm, gb_v)

    def gen_offsets(tt, _):
        p = tt * L
        vu = uidx_v[pl.ds(p, L)]
        vi = iidx_v[pl.ds(p, L)]
        for d in range(D):
            offu_v[pl.ds(d * NPW + p, L)] = vu + (d * V)
            offi_v[pl.ds(d * NPW + p, L)] = vi + (D * V + d * V)
        return 0

    lax.fori_loop(0, NPW // L, gen_offsets, 0)

    for c in range((D * NPW) // CH):
        s = pl.ds(c * CH, CH)
        pltpu.async_copy(flat_hbm.at[offu_v.at[s]], eu_v.at[s], sem)
        pltpu.async_copy(flat_hbm.at[offi_v.at[s]], ei_v.at[s], sem)
    for c in range(NCH):
        s = pl.ds(c * CH, CH)
        pltpu.async_copy(bu_hbm.at[uidx_v.at[s]], bu_v.at[s], sem)
        pltpu.async_copy(bi_hbm.at[iidx_v.at[s]], bi_v.at[s], sem)

    pltpu.make_async_copy(uidx_hbm.at[pl.ds(0, D * NPW)], eu_v, sem).wait()
    pltpu.make_async_copy(uidx_hbm.at[pl.ds(0, D * NPW)], ei_v, sem).wait()
    pltpu.make_async_copy(uidx_hbm.at[pl.ds(0, NPW)], bu_v, sem).wait()
    pltpu.make_async_copy(uidx_hbm.at[pl.ds(0, NPW)], bi_v, sem).wait()

    gb = gb_v[...]

    def block(tt, _):
        p = tt * L
        acc = bu_v[pl.ds(p, L)] + bi_v[pl.ds(p, L)] + gb
        for d in range(D):
            acc = acc + eu_v[pl.ds(d * NPW + p, L)] * ei_v[pl.ds(d * NPW + p, L)]
        out_v[pl.ds(p, L)] = jnp.minimum(jnp.maximum(acc, 1.0), 5.0)
        return 0

    lax.fori_loop(0, NPW // L, block, 0)

    pltpu.sync_copy(out_v, out_hbm.at[pl.ds(base, NPW)])


@jax.jit
def _funk_svd_sc(uidx, iidx, emb_ut, emb_it, bias_u, bias_i, gb16):
    mesh = plsc.VectorSubcoreMesh(
        core_axis_name="c", subcore_axis_name="s",
        num_cores=NC, num_subcores=NS)

    detile = pl.kernel(
        _detile_body,
        out_type=jax.ShapeDtypeStruct((2 * D * V,), jnp.float32),
        mesh=mesh,
        scratch_types=[
            pltpu.VMEM((2 * CHW,), jnp.float32),
            pltpu.SemaphoreType.DMA,
            pltpu.SemaphoreType.DMA,
        ],
        compiler_params=pltpu.CompilerParams(
            needs_layout_passes=False,
            disable_bounds_checks=True,
        ),
    )
    flat = detile(emb_ut, emb_it)

    gather = pl.kernel(
        _gather_body,
        out_type=jax.ShapeDtypeStruct((B,), jnp.float32),
        mesh=mesh,
        scratch_types=[
            pltpu.VMEM((NPW,), jnp.int32),
            pltpu.VMEM((NPW,), jnp.int32),
            pltpu.VMEM((D * NPW,), jnp.int32),
            pltpu.VMEM((D * NPW,), jnp.int32),
            pltpu.VMEM((D * NPW,), jnp.float32),
            pltpu.VMEM((D * NPW,), jnp.float32),
            pltpu.VMEM((NPW,), jnp.float32),
            pltpu.VMEM((NPW,), jnp.float32),
            pltpu.VMEM((L,), jnp.float32),
            pltpu.VMEM((NPW,), jnp.float32),
            pltpu.SemaphoreType.DMA,
        ],
        compiler_params=pltpu.CompilerParams(
            needs_layout_passes=False,
            use_tc_tiling_on_sc=False,
        ),
    )
    return gather(uidx, iidx, flat, bias_u, bias_i, gb16)


def kernel(user_idx, item_idx, emb_u, emb_i, bias_u, bias_i, global_bias):
    uidx = user_idx.astype(jnp.int32)
    iidx = item_idx.astype(jnp.int32)
    gb16 = jnp.broadcast_to(global_bias.astype(jnp.float32), (L,))
    return _funk_svd_sc(uidx, iidx, emb_u.T, emb_i.T, bias_u, bias_i, gb16)
